# Initial kernel scaffold; baseline (speedup 1.0000x reference)
import jax
import jax.numpy as jnp
from jax.experimental import pallas as pl


def kernel(layer_indices, embedding_table):
    raise NotImplementedError("write your pallas kernel here")


# speedup vs baseline: 4.1972x; 4.1972x over previous
import functools

import jax
import jax.numpy as jnp
from jax import lax
from jax.experimental import pallas as pl
from jax.experimental.pallas import tpu as pltpu
from jax.experimental.pallas import tpu_sc as plsc

_HIDDEN = 32
_NW = 32
_CHUNK = 128


def kernel(layer_indices, embedding_table):
    b, s = layer_indices.shape
    n = b * s
    per_w = n // _NW
    n_ch = per_w // _CHUNK
    idx = layer_indices.reshape(_NW, n_ch, _CHUNK)

    mesh = plsc.VectorSubcoreMesh(core_axis_name="c", subcore_axis_name="s")

    @functools.partial(
        pl.kernel,
        mesh=mesh,
        out_type=jax.ShapeDtypeStruct((n, _HIDDEN), jnp.float32),
        scratch_types=[
            pltpu.VMEM((n_ch, _CHUNK), jnp.int32),
            pltpu.VMEM((_CHUNK, _HIDDEN), jnp.float32),
            pltpu.SemaphoreType.DMA,
        ],
        compiler_params=pltpu.CompilerParams(use_tc_tiling_on_sc=False),
    )
    def emb(idx_hbm, table_hbm, out_hbm, idx_v, rows_v, sem):
        wid = lax.axis_index("s") * 2 + lax.axis_index("c")
        pltpu.sync_copy(idx_hbm.at[wid], idx_v)
        base = wid * per_w

        def body(j, carry):
            pltpu.async_copy(table_hbm.at[idx_v.at[j]], rows_v, sem).wait()
            pltpu.sync_copy(rows_v, out_hbm.at[pl.ds(base + j * _CHUNK, _CHUNK)])
            return carry

        lax.fori_loop(0, n_ch, body, 0, unroll=False)

    out = emb(idx, embedding_table)
    return out.reshape(b, s, _HIDDEN)


# speedup vs baseline: 5.1919x; 1.2370x over previous
import functools

import jax
import jax.numpy as jnp
from jax import lax
from jax.experimental import pallas as pl
from jax.experimental.pallas import tpu as pltpu
from jax.experimental.pallas import tpu_sc as plsc

_HIDDEN = 32
_NW = 32
_CHUNK = 128


_NBUF = 4


def kernel(layer_indices, embedding_table):
    b, s = layer_indices.shape
    n = b * s
    per_w = n // _NW
    n_ch = per_w // _CHUNK
    n_grp = n_ch // _NBUF
    idx = layer_indices.reshape(_NW, n_ch, _CHUNK)

    mesh = plsc.VectorSubcoreMesh(core_axis_name="c", subcore_axis_name="s")

    @functools.partial(
        pl.kernel,
        mesh=mesh,
        out_type=jax.ShapeDtypeStruct((n, _HIDDEN), jnp.float32),
        scratch_types=[
            pltpu.VMEM((n_ch, _CHUNK), jnp.int32),
            pltpu.VMEM((_NBUF, _CHUNK, _HIDDEN), jnp.float32),
            [pltpu.SemaphoreType.DMA] * _NBUF,
            [pltpu.SemaphoreType.DMA] * _NBUF,
        ],
        compiler_params=pltpu.CompilerParams(use_tc_tiling_on_sc=False),
    )
    def emb(idx_hbm, table_hbm, out_hbm, idx_v, rows_v, gsems, osems):
        wid = lax.axis_index("s") * 2 + lax.axis_index("c")
        pltpu.sync_copy(idx_hbm.at[wid], idx_v)
        base = wid * per_w

        def gather(j, slot):
            return pltpu.make_async_copy(
                table_hbm.at[idx_v.at[j]], rows_v.at[slot], gsems[slot])

        def write_out(j, slot):
            return pltpu.make_async_copy(
                rows_v.at[slot],
                out_hbm.at[pl.ds(base + j * _CHUNK, _CHUNK)],
                osems[slot])

        for slot in range(_NBUF):
            gather(slot, slot).start()

        def body(g, carry):
            j0 = g * _NBUF
            for slot in range(_NBUF):
                gather(j0 + slot, slot).wait()
                write_out(j0 + slot, slot).start()
            for slot in range(_NBUF):
                write_out(j0 + slot, slot).wait()

                @pl.when(g + 1 < n_grp)
                def _(slot=slot):
                    gather(j0 + slot + _NBUF, slot).start()

            return carry

        lax.fori_loop(0, n_grp, body, 0, unroll=False)

    out = emb(idx, embedding_table)
    return out.reshape(b, s, _HIDDEN)


# speedup vs baseline: 5.2891x; 1.0187x over previous
import functools

import jax
import jax.numpy as jnp
from jax import lax
from jax.experimental import pallas as pl
from jax.experimental.pallas import tpu as pltpu
from jax.experimental.pallas import tpu_sc as plsc

_HIDDEN = 32
_NW = 32
_CHUNK = 128


_NBUF = 8


def kernel(layer_indices, embedding_table):
    b, s = layer_indices.shape
    n = b * s
    per_w = n // _NW
    n_ch = per_w // _CHUNK
    n_grp = n_ch // _NBUF
    idx = layer_indices.reshape(_NW, n_ch, _CHUNK)

    mesh = plsc.VectorSubcoreMesh(core_axis_name="c", subcore_axis_name="s")

    @functools.partial(
        pl.kernel,
        mesh=mesh,
        out_type=jax.ShapeDtypeStruct((n, _HIDDEN), jnp.float32),
        scratch_types=[
            pltpu.VMEM((n_ch, _CHUNK), jnp.int32),
            pltpu.VMEM((_NBUF, _CHUNK, _HIDDEN), jnp.float32),
            [pltpu.SemaphoreType.DMA] * _NBUF,
            [pltpu.SemaphoreType.DMA] * _NBUF,
        ],
        compiler_params=pltpu.CompilerParams(use_tc_tiling_on_sc=False),
    )
    def emb(idx_hbm, table_hbm, out_hbm, idx_v, rows_v, gsems, osems):
        wid = lax.axis_index("s") * 2 + lax.axis_index("c")
        pltpu.sync_copy(idx_hbm.at[wid], idx_v)
        base = wid * per_w

        def gather(j, slot):
            return pltpu.make_async_copy(
                table_hbm.at[idx_v.at[j]], rows_v.at[slot], gsems[slot])

        def write_out(j, slot):
            return pltpu.make_async_copy(
                rows_v.at[slot],
                out_hbm.at[pl.ds(base + j * _CHUNK, _CHUNK)],
                osems[slot])

        for slot in range(_NBUF):
            gather(slot, slot).start()

        def body(g, carry):
            j0 = g * _NBUF
            for slot in range(_NBUF):
                gather(j0 + slot, slot).wait()
                write_out(j0 + slot, slot).start()
            for slot in range(_NBUF):
                write_out(j0 + slot, slot).wait()

                @pl.when(g + 1 < n_grp)
                def _(slot=slot):
                    gather(j0 + slot + _NBUF, slot).start()

            return carry

        lax.fori_loop(0, n_grp, body, 0, unroll=False)

    out = emb(idx, embedding_table)
    return out.reshape(b, s, _HIDDEN)
